# baseline (speedup 1.0000x reference)
import jax
import jax.numpy as jnp
from jax import lax
from jax.experimental import pallas as pl
from jax.experimental.pallas import tpu as pltpu
from jax.experimental.pallas import tpu_sc as plsc

VOCAB = 100000
DIM = 64
B = 16384
X_MAX = 100.0
ALPHA = 0.75

NC = 2
NS = 16
L = 16
NW = NC * NS
BPW = B // NW
GROUPS = BPW // L
CHUNK = 128
NCHUNK = BPW // CHUNK
GPC = CHUNK // L

LN2 = 0.6931471805599453
LN_XMAX = 4.605170185988092
SQRT2 = 1.4142135623730951

_PARAMS = pltpu.CompilerParams(needs_layout_passes=False,
                               use_tc_tiling_on_sc=False)
_MESH = plsc.VectorSubcoreMesh(core_axis_name="c", subcore_axis_name="s",
                               num_cores=NC, num_subcores=NS)


def _ln16(x):
    bits = plsc.bitcast(x, jnp.int32)
    e = (bits >> 23) - 127
    m = plsc.bitcast((bits & 0x007FFFFF) | 0x3F800000, jnp.float32)
    big = m >= SQRT2
    m = jnp.where(big, m * 0.5, m)
    e = jnp.where(big, e + 1, e)
    s = (m - 1.0) / (m + 1.0)
    s2 = s * s
    p = 2.0 * s * (1.0 + s2 * (1.0 / 3.0 + s2 * (1.0 / 5.0 + s2 * (1.0 / 7.0 + s2 * (1.0 / 9.0)))))
    return e.astype(jnp.float32) * LN2 + p


def _dots_body(ii_hbm, jj_hbm, wi_hbm, wj_hbm,
               dots_hbm,
               ii_v, jj_v, wi_rows, wj_rows, dots_v, sems):
    c = lax.axis_index("c")
    s = lax.axis_index("s")
    wid = s * NC + c
    base = wid * BPW

    for k in range(NCHUNK):
        pltpu.sync_copy(ii_hbm.at[pl.ds(base + k * CHUNK, CHUNK)], ii_v.at[k])
        pltpu.sync_copy(jj_hbm.at[pl.ds(base + k * CHUNK, CHUNK)], jj_v.at[k])

    copies = []
    for k in range(NCHUNK):
        sl = pl.ds(k * CHUNK, CHUNK)
        copies.append((
            pltpu.async_copy(wi_hbm.at[ii_v.at[k]], wi_rows.at[sl], sems.at[k]),
            pltpu.async_copy(wj_hbm.at[jj_v.at[k]], wj_rows.at[sl], sems.at[k]),
        ))

    lane = lax.iota(jnp.int32, L)
    zf = jnp.zeros((L,), jnp.float32)

    def group(g, carry):
        pid = lane + g * L
        d0 = d1 = d2 = d3 = zf
        for col in range(0, DIM, 4):
            c0 = jnp.full((L,), col, jnp.int32)
            d0 = d0 + (plsc.load_gather(wi_rows, [pid, c0]) *
                       plsc.load_gather(wj_rows, [pid, c0]))
            c1 = jnp.full((L,), col + 1, jnp.int32)
            d1 = d1 + (plsc.load_gather(wi_rows, [pid, c1]) *
                       plsc.load_gather(wj_rows, [pid, c1]))
            c2 = jnp.full((L,), col + 2, jnp.int32)
            d2 = d2 + (plsc.load_gather(wi_rows, [pid, c2]) *
                       plsc.load_gather(wj_rows, [pid, c2]))
            c3 = jnp.full((L,), col + 3, jnp.int32)
            d3 = d3 + (plsc.load_gather(wi_rows, [pid, c3]) *
                       plsc.load_gather(wj_rows, [pid, c3]))
        dots_v[pl.ds(g * L, L)] = (d0 + d1) + (d2 + d3)
        return carry

    for k in range(NCHUNK):
        for cp in copies[k]:
            cp.wait()
        lax.fori_loop(k * GPC, (k + 1) * GPC, group, 0)

    pltpu.sync_copy(dots_v, dots_hbm.at[pl.ds(base, BPW)])


_dots_call = pl.kernel(
    _dots_body,
    out_type=jax.ShapeDtypeStruct((B,), jnp.float32),
    mesh=_MESH,
    compiler_params=_PARAMS,
    scratch_types=[
        pltpu.VMEM((NCHUNK, CHUNK), jnp.int32),
        pltpu.VMEM((NCHUNK, CHUNK), jnp.int32),
        pltpu.VMEM((BPW, DIM), jnp.float32),
        pltpu.VMEM((BPW, DIM), jnp.float32),
        pltpu.VMEM((BPW,), jnp.float32),
        pltpu.SemaphoreType.DMA((NCHUNK,)),
    ],
)


def _loss_body(ii_hbm, jj_hbm, x_hbm, dots_hbm, bi_hbm, bj_hbm,
               out_hbm,
               ii_v, jj_v, x_v, dots_v, bi_g, bj_g,
               accv, idx16_v, acc_shared, sem):
    c = lax.axis_index("c")
    s = lax.axis_index("s")
    wid = s * NC + c
    base = wid * BPW

    for k in range(NCHUNK):
        pltpu.sync_copy(ii_hbm.at[pl.ds(base + k * CHUNK, CHUNK)], ii_v.at[k])
        pltpu.sync_copy(jj_hbm.at[pl.ds(base + k * CHUNK, CHUNK)], jj_v.at[k])
    pltpu.sync_copy(x_hbm.at[pl.ds(base, BPW)], x_v)
    pltpu.sync_copy(dots_hbm.at[pl.ds(base, BPW)], dots_v)

    copies = []
    for k in range(NCHUNK):
        sl = pl.ds(k * CHUNK, CHUNK)
        copies.append(pltpu.async_copy(bi_hbm.at[ii_v.at[k]], bi_g.at[sl], sem))
        copies.append(pltpu.async_copy(bj_hbm.at[jj_v.at[k]], bj_g.at[sl], sem))
    for cp in copies:
        cp.wait()

    lane = lax.iota(jnp.int32, L)

    def group(g, acc):
        sl = pl.ds(g * L, L)
        dots = dots_v[sl]
        bgi = bi_g[sl]
        bgj = bj_g[sl]
        xg = x_v[sl]
        lnx = _ln16(xg)
        diff = dots + bgi + bgj - lnx
        diff = jnp.where(diff != diff, 0.0, diff)
        diff = jnp.where(jnp.abs(diff) == jnp.inf, 0.0, diff)
        fw = jnp.where(xg > X_MAX, 1.0, jnp.exp(ALPHA * (lnx - LN_XMAX)))
        return acc + 0.5 * fw * diff * diff

    acc = lax.fori_loop(0, GROUPS, group, jnp.zeros((L,), jnp.float32))

    accv[...] = acc
    idx16_v[...] = lane

    @pl.when(s == 0)
    def _():
        pltpu.sync_copy(accv, acc_shared)

    plsc.subcore_barrier()

    @pl.when(s != 0)
    def _():
        pltpu.sync_copy(accv, acc_shared.at[idx16_v], add=True)

    plsc.subcore_barrier()

    @pl.when(s == 0)
    def _():
        pltpu.sync_copy(acc_shared, accv)
        pltpu.sync_copy(accv, out_hbm.at[c])


_loss_call = pl.kernel(
    _loss_body,
    out_type=jax.ShapeDtypeStruct((NC, L), jnp.float32),
    mesh=_MESH,
    compiler_params=_PARAMS,
    scratch_types=[
        pltpu.VMEM((NCHUNK, CHUNK), jnp.int32),
        pltpu.VMEM((NCHUNK, CHUNK), jnp.int32),
        pltpu.VMEM((BPW,), jnp.float32),
        pltpu.VMEM((BPW,), jnp.float32),
        pltpu.VMEM((BPW,), jnp.float32),
        pltpu.VMEM((BPW,), jnp.float32),
        pltpu.VMEM((L,), jnp.float32),
        pltpu.VMEM((L,), jnp.int32),
        pltpu.VMEM_SHARED((L,), jnp.float32),
        pltpu.SemaphoreType.DMA,
    ],
)


@jax.jit
def kernel(i_indices, j_indices, x_ij, wi, wj, bi, bj):
    ii = i_indices.astype(jnp.int32)
    jj = j_indices.astype(jnp.int32)
    dots = _dots_call(ii, jj, wi, wj)
    part = _loss_call(ii, jj, x_ij, dots, bi.reshape(VOCAB), bj.reshape(VOCAB))
    return jnp.sum(part) * (1.0 / B)


# speedup vs baseline: 1.0690x; 1.0690x over previous
import jax
import jax.numpy as jnp
from jax import lax
from jax.experimental import pallas as pl
from jax.experimental.pallas import tpu as pltpu
from jax.experimental.pallas import tpu_sc as plsc

VOCAB = 100000
DIM = 64
B = 16384
X_MAX = 100.0
ALPHA = 0.75

NC = 2
NS = 16
L = 16
NW = NC * NS
BPW = B // NW
GROUPS = BPW // L
CHUNK = 128
NCHUNK = BPW // CHUNK

LN2 = 0.6931471805599453
LN_XMAX = 4.605170185988092
SQRT2 = 1.4142135623730951


def _ln16(x):
    bits = plsc.bitcast(x, jnp.int32)
    e = (bits >> 23) - 127
    m = plsc.bitcast((bits & 0x007FFFFF) | 0x3F800000, jnp.float32)
    big = m >= SQRT2
    m = jnp.where(big, m * 0.5, m)
    e = jnp.where(big, e + 1, e)
    s = (m - 1.0) / (m + 1.0)
    s2 = s * s
    p = 2.0 * s * (1.0 + s2 * (1.0 / 3.0 + s2 * (1.0 / 5.0 + s2 * (1.0 / 7.0 + s2 * (1.0 / 9.0)))))
    return e.astype(jnp.float32) * LN2 + p


def _glove_body(ii_hbm, jj_hbm, x_hbm, wi_hbm, wj_hbm, bi_hbm, bj_hbm,
                out_hbm,
                ii_v, jj_v, x_v, wi_rows, wj_rows, bi_g, bj_g,
                accv, idx16_v, acc_shared, sem):
    c = lax.axis_index("c")
    s = lax.axis_index("s")
    wid = s * NC + c
    base = wid * BPW

    for k in range(NCHUNK):
        pltpu.sync_copy(ii_hbm.at[pl.ds(base + k * CHUNK, CHUNK)], ii_v.at[k])
        pltpu.sync_copy(jj_hbm.at[pl.ds(base + k * CHUNK, CHUNK)], jj_v.at[k])
    pltpu.sync_copy(x_hbm.at[pl.ds(base, BPW)], x_v)

    copies = []
    for k in range(NCHUNK):
        copies.append(pltpu.async_copy(
            wi_hbm.at[ii_v.at[k]], wi_rows.at[pl.ds(k * CHUNK, CHUNK)], sem))
        copies.append(pltpu.async_copy(
            wj_hbm.at[jj_v.at[k]], wj_rows.at[pl.ds(k * CHUNK, CHUNK)], sem))
        copies.append(pltpu.async_copy(
            bi_hbm.at[ii_v.at[k]], bi_g.at[pl.ds(k * CHUNK, CHUNK)], sem))
        copies.append(pltpu.async_copy(
            bj_hbm.at[jj_v.at[k]], bj_g.at[pl.ds(k * CHUNK, CHUNK)], sem))
    for cp in copies:
        cp.wait()

    lane = lax.iota(jnp.int32, L)
    zero16 = jnp.zeros((L,), jnp.int32)

    def group(g, acc):
        pid = lane + g * L
        dots = jnp.zeros((L,), jnp.float32)
        for col in range(DIM):
            ci = jnp.full((L,), col, jnp.int32)
            a = plsc.load_gather(wi_rows, [pid, ci])
            b = plsc.load_gather(wj_rows, [pid, ci])
            dots = dots + a * b
        bgi = bi_g[pl.ds(g * L, L)]
        bgj = bj_g[pl.ds(g * L, L)]
        xg = x_v[pl.ds(g * L, L)]
        lnx = _ln16(xg)
        diff = dots + bgi + bgj - lnx
        diff = jnp.where(diff != diff, 0.0, diff)
        diff = jnp.where(jnp.abs(diff) == jnp.inf, 0.0, diff)
        fw = jnp.where(xg > X_MAX, 1.0, jnp.exp(ALPHA * (lnx - LN_XMAX)))
        return acc + 0.5 * fw * diff * diff

    acc = lax.fori_loop(0, GROUPS, group, jnp.zeros((L,), jnp.float32))

    accv[...] = acc
    idx16_v[...] = lane

    @pl.when(s == 0)
    def _():
        pltpu.sync_copy(accv, acc_shared)

    plsc.subcore_barrier()

    @pl.when(s != 0)
    def _():
        pltpu.sync_copy(accv, acc_shared.at[idx16_v], add=True)

    plsc.subcore_barrier()

    @pl.when(s == 0)
    def _():
        pltpu.sync_copy(acc_shared, accv)
        pltpu.sync_copy(accv, out_hbm.at[c])


_glove_call = pl.kernel(
    _glove_body,
    out_type=jax.ShapeDtypeStruct((NC, L), jnp.float32),
    mesh=plsc.VectorSubcoreMesh(core_axis_name="c", subcore_axis_name="s",
                                num_cores=NC, num_subcores=NS),
    compiler_params=pltpu.CompilerParams(needs_layout_passes=False,
                                         use_tc_tiling_on_sc=False),
    scratch_types=[
        pltpu.VMEM((NCHUNK, CHUNK), jnp.int32),
        pltpu.VMEM((NCHUNK, CHUNK), jnp.int32),
        pltpu.VMEM((BPW,), jnp.float32),
        pltpu.VMEM((BPW, DIM), jnp.float32),
        pltpu.VMEM((BPW, DIM), jnp.float32),
        pltpu.VMEM((BPW,), jnp.float32),
        pltpu.VMEM((BPW,), jnp.float32),
        pltpu.VMEM((L,), jnp.float32),
        pltpu.VMEM((L,), jnp.int32),
        pltpu.VMEM_SHARED((L,), jnp.float32),
        pltpu.SemaphoreType.DMA,
    ],
)


@jax.jit
def kernel(i_indices, j_indices, x_ij, wi, wj, bi, bj):
    part = _glove_call(i_indices.astype(jnp.int32), j_indices.astype(jnp.int32),
                       x_ij, wi, wj, bi.reshape(VOCAB), bj.reshape(VOCAB))
    return jnp.sum(part) * (1.0 / B)
